# Initial kernel scaffold; baseline (speedup 1.0000x reference)
import jax
import jax.numpy as jnp
from jax.experimental import pallas as pl


def kernel(x, targets, embed_table):
    raise NotImplementedError("write your pallas kernel here")


# speedup vs baseline: 1.2009x; 1.2009x over previous
import jax
import jax.numpy as jnp
from jax import lax
from jax.experimental import pallas as pl
from jax.experimental.pallas import tpu as pltpu
from jax.experimental.pallas import tpu_sc as plsc

V = 1000
B = 16384
NC = 2
NS = 16
L = 16
NW = NC * NS
BPW = B // NW
CH = 64
G = 128


def _lse_body(t_ref, lse_ref):
    t = t_ref[...]
    m = jnp.max(t, axis=1, keepdims=True)
    lse_ref[...] = m + jnp.log(jnp.sum(jnp.exp(t - m), axis=1, keepdims=True))


def _loss_body(p_ref, loss_ref):
    loss_ref[...] = jnp.sum(p_ref[...]).reshape(1, 1) * (1.0 / B)


def _sc_body(table_h, x_h, tgt_h, lse_h, out_h, part_h,
             x_v, t_v, ls_v, rows_v, acc_v, sem, sem2):
    c = lax.axis_index("c")
    s = lax.axis_index("s")
    wid = s * NC + c
    base = wid * BPW
    pltpu.sync_copy(x_h.at[pl.ds(base, BPW)], x_v)
    pltpu.sync_copy(tgt_h.at[pl.ds(base, BPW)], t_v)
    handles = []
    for g in range(BPW // G):
        sl = pl.ds(g * G, G)
        handles.append(
            pltpu.async_copy(lse_h.at[x_v.at[sl]], ls_v.at[sl], sem2))
    for h in handles:
        h.wait()
    iota = lax.iota(jnp.int32, L)
    acc = jnp.zeros((L,), jnp.float32)
    for ch in range(BPW // CH):
        pltpu.async_copy(table_h.at[x_v.at[pl.ds(ch * CH, CH)]], rows_v,
                         sem).wait()
        pltpu.sync_copy(rows_v, out_h.at[pl.ds(base + ch * CH, CH)])
        for j in range(CH // L):
            off = ch * CH + j * L
            tl = plsc.load_gather(rows_v, [iota + (j * L), t_v[pl.ds(off, L)]])
            acc = acc + ls_v[pl.ds(off, L)] - tl
    acc_v[...] = acc
    pltpu.sync_copy(acc_v, part_h.at[wid])


def kernel(x, targets, embed_table):
    x = x.astype(jnp.int32)
    targets = targets.astype(jnp.int32)
    table = embed_table.astype(jnp.float32)

    lse = pl.pallas_call(
        _lse_body,
        out_shape=jax.ShapeDtypeStruct((V, 1), jnp.float32),
    )(table)

    sc_call = pl.kernel(
        _sc_body,
        mesh=plsc.VectorSubcoreMesh(core_axis_name="c", subcore_axis_name="s"),
        compiler_params=pltpu.CompilerParams(use_tc_tiling_on_sc=False,
                                              needs_layout_passes=False),
        out_type=[
            jax.ShapeDtypeStruct((B, V), jnp.float32),
            jax.ShapeDtypeStruct((NW, L), jnp.float32),
        ],
        scratch_types=[
            pltpu.VMEM((BPW,), jnp.int32),
            pltpu.VMEM((BPW,), jnp.int32),
            pltpu.VMEM((BPW,), jnp.float32),
            pltpu.VMEM((CH, V), jnp.float32),
            pltpu.VMEM((L,), jnp.float32),
            pltpu.SemaphoreType.DMA,
            pltpu.SemaphoreType.DMA,
        ],
    )
    logits, partials = sc_call(table, x, targets, lse.reshape(V))

    loss = pl.pallas_call(
        _loss_body,
        out_shape=jax.ShapeDtypeStruct((1, 1), jnp.float32),
    )(partials)
    return logits, loss.reshape(())


# speedup vs baseline: 1.2213x; 1.0170x over previous
import jax
import jax.numpy as jnp
from jax import lax
from jax.experimental import pallas as pl
from jax.experimental.pallas import tpu as pltpu
from jax.experimental.pallas import tpu_sc as plsc

V = 1000
B = 16384
NC = 2
NS = 16
L = 16
NW = NC * NS
BPW = B // NW
CH = 64
G = 128


def _lse_body(t_ref, lse_ref):
    t = t_ref[...]
    m = jnp.max(t, axis=1, keepdims=True)
    lse_ref[...] = m + jnp.log(jnp.sum(jnp.exp(t - m), axis=1, keepdims=True))


def _loss_body(p_ref, loss_ref):
    loss_ref[...] = jnp.sum(p_ref[...]).reshape(1, 1) * (1.0 / B)


def _sc_body(table_h, x_h, tgt_h, lse_h, out_h, part_h,
             x_v, t_v, ls_v, rows0_v, rows1_v, acc_v,
             semg0, semg1, semw0, semw1, sem2):
    c = lax.axis_index("c")
    s = lax.axis_index("s")
    wid = s * NC + c
    base = wid * BPW
    pltpu.sync_copy(x_h.at[pl.ds(base, BPW)], x_v)
    pltpu.sync_copy(tgt_h.at[pl.ds(base, BPW)], t_v)
    bufs = (rows0_v, rows1_v)
    semg = (semg0, semg1)
    semw = (semw0, semw1)

    def gather(ch):
        b = ch % 2
        return pltpu.async_copy(
            table_h.at[x_v.at[pl.ds(ch * CH, CH)]], bufs[b], semg[b])

    NCH = BPW // CH
    hg = [None, None]
    hw = [None, None]
    hg[0] = gather(0)
    hs = []
    for g in range(BPW // G):
        sl = pl.ds(g * G, G)
        hs.append(pltpu.async_copy(lse_h.at[x_v.at[sl]], ls_v.at[sl], sem2))
    for h in hs:
        h.wait()
    iota = lax.iota(jnp.int32, L)
    acc = jnp.zeros((L,), jnp.float32)
    for ch in range(NCH):
        b = ch % 2
        hg[b].wait()
        if ch + 1 < NCH:
            b2 = (ch + 1) % 2
            if hw[b2] is not None:
                hw[b2].wait()
            hg[b2] = gather(ch + 1)
        hw[b] = pltpu.async_copy(bufs[b], out_h.at[pl.ds(base + ch * CH, CH)],
                                 semw[b])
        for j in range(CH // L):
            off = ch * CH + j * L
            tl = plsc.load_gather(bufs[b],
                                  [iota + (j * L), t_v[pl.ds(off, L)]])
            acc = acc + ls_v[pl.ds(off, L)] - tl
    for h in hw:
        if h is not None:
            h.wait()
    acc_v[...] = acc
    pltpu.sync_copy(acc_v, part_h.at[wid])


def kernel(x, targets, embed_table):
    x = x.astype(jnp.int32)
    targets = targets.astype(jnp.int32)
    table = embed_table.astype(jnp.float32)

    lse = pl.pallas_call(
        _lse_body,
        out_shape=jax.ShapeDtypeStruct((V, 1), jnp.float32),
    )(table)

    sc_call = pl.kernel(
        _sc_body,
        mesh=plsc.VectorSubcoreMesh(core_axis_name="c", subcore_axis_name="s"),
        compiler_params=pltpu.CompilerParams(use_tc_tiling_on_sc=False,
                                              needs_layout_passes=False),
        out_type=[
            jax.ShapeDtypeStruct((B, V), jnp.float32),
            jax.ShapeDtypeStruct((NW, L), jnp.float32),
        ],
        scratch_types=[
            pltpu.VMEM((BPW,), jnp.int32),
            pltpu.VMEM((BPW,), jnp.int32),
            pltpu.VMEM((BPW,), jnp.float32),
            pltpu.VMEM((CH, V), jnp.float32),
            pltpu.VMEM((CH, V), jnp.float32),
            pltpu.VMEM((L,), jnp.float32),
            pltpu.SemaphoreType.DMA,
            pltpu.SemaphoreType.DMA,
            pltpu.SemaphoreType.DMA,
            pltpu.SemaphoreType.DMA,
            pltpu.SemaphoreType.DMA,
        ],
    )
    logits, partials = sc_call(table, x, targets, lse.reshape(V))

    loss = pl.pallas_call(
        _loss_body,
        out_shape=jax.ShapeDtypeStruct((1, 1), jnp.float32),
    )(partials)
    return logits, loss.reshape(())


# speedup vs baseline: 1.3189x; 1.0799x over previous
import jax
import jax.numpy as jnp
from jax import lax
from jax.experimental import pallas as pl
from jax.experimental.pallas import tpu as pltpu
from jax.experimental.pallas import tpu_sc as plsc

V = 1000
VP = 1024
B = 16384
NC = 2
NS = 16
L = 16
NW = NC * NS
BPW = B // NW
CH = 32
G = 128


def _lse_body(t_ref, lse_ref):
    t = t_ref[...]
    m = jnp.max(t, axis=1, keepdims=True)
    lse_ref[...] = m + jnp.log(jnp.sum(jnp.exp(t - m), axis=1, keepdims=True))


def _loss_body(p_ref, loss_ref):
    loss_ref[...] = jnp.sum(p_ref[...]).reshape(1, 1) * (1.0 / B)


def _sc_body(table3_h, tflat_h, x_h, tgt_h, lse_h, out_h, part_h,
             x_v, t_v, fidx_v, tl_v, ls_v, rows0_v, rows1_v, acc_v,
             semg0, semg1, semw0, semw1, sem2):
    c = lax.axis_index("c")
    s = lax.axis_index("s")
    wid = s * NC + c
    base = wid * BPW
    pltpu.sync_copy(x_h.at[pl.ds(base, BPW)], x_v)
    pltpu.sync_copy(tgt_h.at[pl.ds(base, BPW)], t_v)
    for i in range(BPW // L):
        sl = pl.ds(i * L, L)
        fidx_v[sl] = x_v[sl] * V + t_v[sl]
    hs = []
    for g in range(BPW // G):
        sl = pl.ds(g * G, G)
        hs.append(
            pltpu.async_copy(tflat_h.at[fidx_v.at[sl]], tl_v.at[sl], sem2))
        hs.append(
            pltpu.async_copy(lse_h.at[x_v.at[sl]], ls_v.at[sl], sem2))

    bufs = (rows0_v, rows1_v)
    semg = (semg0, semg1)
    semw = (semw0, semw1)

    def gather(ch):
        b = ch % 2
        return pltpu.async_copy(
            table3_h.at[x_v.at[pl.ds(ch * CH, CH)]], bufs[b], semg[b])

    NCH = BPW // CH
    hg = [None, None]
    hw = [None, None]
    hg[0] = gather(0)
    for ch in range(NCH):
        b = ch % 2
        hg[b].wait()
        if ch + 1 < NCH:
            b2 = (ch + 1) % 2
            if hw[b2] is not None:
                hw[b2].wait()
            hg[b2] = gather(ch + 1)
        hw[b] = pltpu.async_copy(bufs[b],
                                 out_h.at[pl.ds(base + ch * CH, CH)], semw[b])
    for h in hs:
        h.wait()
    acc = jnp.zeros((L,), jnp.float32)
    for i in range(BPW // L):
        sl = pl.ds(i * L, L)
        acc = acc + ls_v[sl] - tl_v[sl]
    acc_v[...] = acc
    for h in hw:
        if h is not None:
            h.wait()
    pltpu.sync_copy(acc_v, part_h.at[pl.ds(wid * L, L)])


def kernel(x, targets, embed_table):
    x = x.astype(jnp.int32)
    targets = targets.astype(jnp.int32)
    table = embed_table.astype(jnp.float32)

    lse = pl.pallas_call(
        _lse_body,
        out_shape=jax.ShapeDtypeStruct((V, 1), jnp.float32),
    )(table)

    table3 = jnp.pad(table, ((0, 0), (0, VP - V))).reshape(V, 8, 128)
    tflat = table.reshape(V * V)

    sc_call = pl.kernel(
        _sc_body,
        mesh=plsc.VectorSubcoreMesh(core_axis_name="c", subcore_axis_name="s"),
        out_type=[
            jax.ShapeDtypeStruct((B, 8, 128), jnp.float32),
            jax.ShapeDtypeStruct((NW * L,), jnp.float32),
        ],
        scratch_types=[
            pltpu.VMEM((BPW,), jnp.int32),
            pltpu.VMEM((BPW,), jnp.int32),
            pltpu.VMEM((BPW,), jnp.int32),
            pltpu.VMEM((BPW,), jnp.float32),
            pltpu.VMEM((BPW,), jnp.float32),
            pltpu.VMEM((CH, 8, 128), jnp.float32),
            pltpu.VMEM((CH, 8, 128), jnp.float32),
            pltpu.VMEM((L,), jnp.float32),
            pltpu.SemaphoreType.DMA,
            pltpu.SemaphoreType.DMA,
            pltpu.SemaphoreType.DMA,
            pltpu.SemaphoreType.DMA,
            pltpu.SemaphoreType.DMA,
        ],
    )
    out3, partials = sc_call(table3, tflat, x, targets, lse.reshape(V))

    logits = out3.reshape(B, VP)[:, :V]
    loss = pl.pallas_call(
        _loss_body,
        out_shape=jax.ShapeDtypeStruct((1, 1), jnp.float32),
    )(partials.reshape(NW, L))
    return logits, loss.reshape(())
